# baseline (speedup 1.0000x reference)
import functools

import jax
import jax.numpy as jnp
from jax import lax
from jax.experimental import pallas as pl
from jax.experimental.pallas import tpu as pltpu
from jax.experimental.pallas import tpu_sc as plsc

_VOCAB = 100000
_N_PREFIX = 100
_DIM = 64
_LANES = 16
_NC = 2
_NS = 16
_NW = _NC * _NS
_B = 16384
_L = 50
_CHUNK = 256
_SUB = 128
_NBUF = 2
_BQ = _B // _CHUNK
_DT = _DIM // 8
_NBT = _CHUNK // _SUB
_PITCH = _SUB + 1


@functools.cache
def _make_gather(n_idx):
    n_chunks = n_idx // _CHUNK
    n_per_w = n_chunks // _NW
    n_rows = n_idx * _DIM // _SUB
    mesh = plsc.VectorSubcoreMesh(core_axis_name="c", subcore_axis_name="s",
                                  num_cores=_NC, num_subcores=_NS)

    @functools.partial(
        pl.kernel,
        out_type=jax.ShapeDtypeStruct((n_rows, _SUB), jnp.float32),
        mesh=mesh,
        compiler_params=pltpu.CompilerParams(use_tc_tiling_on_sc=False,
                                             needs_layout_passes=False),
        scratch_types=[
            pltpu.VMEM((_NBUF, _CHUNK), jnp.int32),
            pltpu.VMEM((_NBUF, _CHUNK), jnp.int32),
            pltpu.VMEM((_NBUF, _CHUNK, _DIM), jnp.float32),
            pltpu.VMEM((_NBUF, _NBT * _DIM, _PITCH), jnp.float32),
            pltpu.VMEM((_N_PREFIX, _DIM), jnp.float32),
            pltpu.SemaphoreType.DMA,
            pltpu.SemaphoreType.DMA,
            pltpu.SemaphoreType.DMA,
        ],
    )
    def gather_kernel(w_hbm, wn_hbm, idx_hbm, out_hbm,
                      idxo_v, idxc_v, rows_v, stage_v, wn_v,
                      gsem, ssem0, ssem1):
        wid = lax.axis_index("s") * _NC + lax.axis_index("c")
        ssems = (ssem0, ssem1)
        pltpu.sync_copy(wn_hbm, wn_v)
        lanes16 = lax.iota(jnp.int32, _LANES)
        dg_vecs = [dg * _LANES + lanes16 for dg in range(_DIM // _LANES)]

        def offsets(c):
            l = c // _BQ
            bq = c % _BQ
            p0 = l * _B + bq * _CHUNK
            r0 = (l * _DT * (_B // _SUB) + bq * _NBT) * 8
            return p0, r0

        def store_dmas(stage, r0, sem):
            return [
                pltpu.make_async_copy(
                    stage.at[pl.ds(bt * _DIM + dt * 8, 8), pl.ds(0, _SUB)],
                    out_hbm.at[pl.ds(r0 + dt * (_B // _SUB) * 8 + bt * 8, 8)],
                    sem,
                )
                for dt in range(_DT)
                for bt in range(_NBT)
            ]

        def do_chunk(g, b, first):
            c = wid * n_per_w + g
            p0, r0 = offsets(c)
            idxo = idxo_v.at[b]
            idxc = idxc_v.at[b]
            rows = rows_v.at[b]
            stage = stage_v.at[b]
            pltpu.sync_copy(idx_hbm.at[pl.ds(p0, _CHUNK)], idxo)

            @plsc.parallel_loop(0, _CHUNK // _LANES, unroll=4,
                                carry=jnp.bool_(False))
            def has_prefix(j, acc):
                v = idxo[pl.ds(j * _LANES, _LANES)]
                m = v >= _VOCAB
                idxc[pl.ds(j * _LANES, _LANES)] = jnp.where(m, _VOCAB - 1, v)
                return acc | jnp.any(m)

            copies = [
                pltpu.async_copy(
                    w_hbm.at[idxc.at[pl.ds(k * _SUB, _SUB)]],
                    rows.at[pl.ds(k * _SUB, _SUB)],
                    gsem,
                )
                for k in range(_NBT)
            ]
            for cp in copies:
                cp.wait()

            @pl.when(has_prefix)
            def _patch():
                def patch_slice(j, acc):
                    v = idxo[pl.ds(j * _LANES, _LANES)]
                    m = v >= _VOCAB

                    @pl.when(jnp.any(m))
                    def _do_patch():
                        e = jnp.where(m, v - _VOCAB, 0)
                        rows16 = j * _LANES + lanes16

                        def col_body(d, cc):
                            colv = jnp.full((_LANES,), d, jnp.int32)
                            vals = plsc.load_gather(wn_v, [e, colv], mask=m)
                            plsc.store_scatter(rows, [rows16, colv], vals,
                                               mask=m)
                            return cc

                        lax.fori_loop(0, _DIM, col_body, jnp.int32(0))

                    return acc

                lax.fori_loop(0, _CHUNK // _LANES, patch_slice, jnp.int32(0))

            @pl.when(jnp.logical_not(first))
            def _drain_prev():
                for cp in store_dmas(stage, r0, ssems[b]):
                    cp.wait()

            @plsc.parallel_loop(0, _CHUNK, unroll=16)
            def _tr(r):
                bt = lax.shift_right_logical(r, 7)
                bi = lax.bitwise_and(r, _SUB - 1)
                biv = jnp.full((_LANES,), bi, jnp.int32)
                pbase = bt * _DIM
                for dg in range(_DIM // _LANES):
                    pv = pbase + dg_vecs[dg]
                    vals = rows[r, pl.ds(dg * _LANES, _LANES)]
                    plsc.store_scatter(stage, [pv, biv], vals)

            for cp in store_dmas(stage, r0, ssems[b]):
                cp.start()

        def step_body(s, carry):
            for b in range(_NBUF):
                do_chunk(s * _NBUF + b, b, s < 1)
            return carry

        lax.fori_loop(0, n_per_w // _NBUF, step_body, jnp.int32(0))

        for b in range(_NBUF):
            g = (n_per_w // _NBUF - 1) * _NBUF + b
            _, r0 = offsets(wid * n_per_w + g)
            for cp in store_dmas(stage_v.at[b], r0, ssems[b]):
                cp.wait()

    return gather_kernel


def kernel(input, W, W_new):
    b, l = input.shape
    idx = input.T.reshape(-1).astype(jnp.int32)
    flat = _make_gather(idx.shape[0])(W, W_new, idx)
    out5 = flat.reshape(l, _DIM // 8, b // _SUB, 8, _SUB)
    return out5.transpose(2, 4, 0, 1, 3).reshape(b, l, _DIM)


# speedup vs baseline: 1.3306x; 1.3306x over previous
import functools

import jax
import jax.numpy as jnp
from jax import lax
from jax.experimental import pallas as pl
from jax.experimental.pallas import tpu as pltpu
from jax.experimental.pallas import tpu_sc as plsc

_VOCAB = 100000
_N_PREFIX = 100
_DIM = 64
_LANES = 16
_NC = 2
_NS = 16
_NW = _NC * _NS
_B = 16384
_L = 50
_CHUNK = 256
_SUB = 128
_NBUF = 2
_BQ = _B // _CHUNK
_DT = _DIM // 8
_NBT = _CHUNK // _SUB
_PITCH = _SUB + 1


@functools.cache
def _make_gather(n_idx):
    n_chunks = n_idx // _CHUNK
    n_per_w = n_chunks // _NW
    n_rows = n_idx * _DIM // _SUB
    mesh = plsc.VectorSubcoreMesh(core_axis_name="c", subcore_axis_name="s",
                                  num_cores=_NC, num_subcores=_NS)

    @functools.partial(
        pl.kernel,
        out_type=jax.ShapeDtypeStruct((n_rows, _SUB), jnp.float32),
        mesh=mesh,
        compiler_params=pltpu.CompilerParams(use_tc_tiling_on_sc=False,
                                             needs_layout_passes=False),
        scratch_types=[
            pltpu.VMEM((_NBUF, _CHUNK), jnp.int32),
            pltpu.VMEM((_NBUF, _CHUNK), jnp.int32),
            pltpu.VMEM((_NBUF, _CHUNK, _DIM), jnp.float32),
            pltpu.VMEM((_NBUF, _NBT * _DIM, _PITCH), jnp.float32),
            pltpu.VMEM((_N_PREFIX, _DIM), jnp.float32),
            pltpu.SemaphoreType.DMA,
            pltpu.SemaphoreType.DMA,
            pltpu.SemaphoreType.DMA,
            pltpu.SemaphoreType.DMA,
            pltpu.SemaphoreType.DMA,
            pltpu.SemaphoreType.DMA,
        ],
    )
    def gather_kernel(w_hbm, wn_hbm, idx_hbm, out_hbm,
                      idxo_v, idxc_v, rows_v, stage_v, wn_v,
                      isem0, isem1, gsem0, gsem1, ssem0, ssem1):
        wid = lax.axis_index("s") * _NC + lax.axis_index("c")
        isems = (isem0, isem1)
        gsems = (gsem0, gsem1)
        ssems = (ssem0, ssem1)
        pltpu.sync_copy(wn_hbm, wn_v)
        lanes16 = lax.iota(jnp.int32, _LANES)
        dg_vecs = [dg * _LANES + lanes16 for dg in range(_DIM // _LANES)]
        c_base = wid * n_per_w

        def offsets(c):
            l = c // _BQ
            bq = c % _BQ
            p0 = l * _B + bq * _CHUNK
            r0 = (l * _DT * (_B // _SUB) + bq * _NBT) * 8
            return p0, r0

        def idx_copy(g, b):
            c = c_base + jnp.minimum(g, n_per_w - 1)
            p0, _ = offsets(c)
            return pltpu.make_async_copy(
                idx_hbm.at[pl.ds(p0, _CHUNK)], idxo_v.at[b], isems[b])

        def gather_copies(g, b):
            idxc = idxc_v.at[b]
            rows = rows_v.at[b]
            return [
                pltpu.async_copy(
                    w_hbm.at[idxc.at[pl.ds(k * _SUB, _SUB)]],
                    rows.at[pl.ds(k * _SUB, _SUB)],
                    gsems[b],
                )
                for k in range(_NBT)
            ]

        def gather_drain(b):
            for k in range(_NBT):
                pltpu.make_async_copy(
                    w_hbm.at[pl.ds(0, _SUB)],
                    rows_v.at[b].at[pl.ds(k * _SUB, _SUB)],
                    gsems[b]).wait()

        def store_dmas(b, r0):
            stage = stage_v.at[b]
            return [
                pltpu.make_async_copy(
                    stage.at[pl.ds(bt * _DIM + dt * 8, 8), pl.ds(0, _SUB)],
                    out_hbm.at[pl.ds(r0 + dt * (_B // _SUB) * 8 + bt * 8, 8)],
                    ssems[b],
                )
                for dt in range(_DT)
                for bt in range(_NBT)
            ]

        def clamp(b):
            idxo = idxo_v.at[b]
            idxc = idxc_v.at[b]

            @plsc.parallel_loop(0, _CHUNK // _LANES, unroll=4,
                                carry=jnp.bool_(False))
            def has_prefix(j, acc):
                v = idxo[pl.ds(j * _LANES, _LANES)]
                m = v >= _VOCAB
                idxc[pl.ds(j * _LANES, _LANES)] = jnp.where(m, _VOCAB - 1, v)
                return acc | jnp.any(m)

            return has_prefix

        def patch(b, has_prefix):
            idxo = idxo_v.at[b]
            rows = rows_v.at[b]

            @pl.when(has_prefix)
            def _patch():
                def patch_slice(j, acc):
                    v = idxo[pl.ds(j * _LANES, _LANES)]
                    m = v >= _VOCAB

                    @pl.when(jnp.any(m))
                    def _do_patch():
                        e = jnp.where(m, v - _VOCAB, 0)
                        rows16 = j * _LANES + lanes16

                        def col_body(d, cc):
                            colv = jnp.full((_LANES,), d, jnp.int32)
                            vals = plsc.load_gather(wn_v, [e, colv], mask=m)
                            plsc.store_scatter(rows, [rows16, colv], vals,
                                               mask=m)
                            return cc

                        lax.fori_loop(0, _DIM, col_body, jnp.int32(0))

                    return acc

                lax.fori_loop(0, _CHUNK // _LANES, patch_slice, jnp.int32(0))

        def transpose(b):
            rows = rows_v.at[b]
            stage = stage_v.at[b]

            @plsc.parallel_loop(0, _CHUNK, unroll=8)
            def _tr(r):
                bt = lax.shift_right_logical(r, 7)
                bi = lax.bitwise_and(r, _SUB - 1)
                biv = jnp.full((_LANES,), bi, jnp.int32)
                pbase = bt * _DIM
                for dg in range(_DIM // _LANES):
                    pv = pbase + dg_vecs[dg]
                    vals = rows[r, pl.ds(dg * _LANES, _LANES)]
                    plsc.store_scatter(stage, [pv, biv], vals)

        idx_copy(0, 0).start()
        idx_copy(1, 1).start()
        idx_copy(0, 0).wait()
        hp0_init = clamp(0)
        for cp in gather_copies(0, 0):
            cp.start()

        def step_body(s, hp0):
            def half(g, b, hp_b, first):
                gather_drain(b)
                patch(b, hp_b)
                idx_copy(g + 2, b).start()
                idx_copy(g + 1, 1 - b).wait()
                hp_next = clamp(1 - b)
                for cp in gather_copies(g + 1, 1 - b):
                    cp.start()
                _, r0 = offsets(c_base + g)

                @pl.when(jnp.logical_not(first))
                def _drain_prev():
                    for cp in store_dmas(b, r0):
                        cp.wait()

                transpose(b)
                for cp in store_dmas(b, r0):
                    cp.start()
                return hp_next

            g0 = s * _NBUF
            hp1 = half(g0, 0, hp0, s < 1)
            hp0_next = half(g0 + 1, 1, hp1, s < 1)
            return hp0_next

        lax.fori_loop(0, n_per_w // _NBUF, step_body, hp0_init)

        idx_copy(n_per_w + 1, 1).wait()
        gather_drain(0)
        for b in range(_NBUF):
            g = n_per_w - _NBUF + b
            _, r0 = offsets(c_base + g)
            for cp in store_dmas(b, r0):
                cp.wait()

    return gather_kernel


def kernel(input, W, W_new):
    b, l = input.shape
    idx = input.T.reshape(-1).astype(jnp.int32)
    flat = _make_gather(idx.shape[0])(W, W_new, idx)
    out5 = flat.reshape(l, _DIM // 8, b // _SUB, 8, _SUB)
    return out5.transpose(2, 4, 0, 1, 3).reshape(b, l, _DIM)


# speedup vs baseline: 1.5816x; 1.1886x over previous
import functools

import jax
import jax.numpy as jnp
from jax import lax
from jax.experimental import pallas as pl
from jax.experimental.pallas import tpu as pltpu
from jax.experimental.pallas import tpu_sc as plsc

_VOCAB = 100000
_N_PREFIX = 100
_DIM = 64
_LANES = 16
_NC = 2
_NS = 16
_NW = _NC * _NS
_B = 16384
_L = 50
_CHUNK = 256
_SUB = 128
_NBUF = 2
_BQ = _B // _CHUNK
_DT = _DIM // 8
_NBT = _CHUNK // _SUB
_PITCH = _SUB + 1


@functools.cache
def _make_gather(n_idx):
    n_chunks = n_idx // _CHUNK
    n_per_w = n_chunks // _NW
    n_rows = n_idx * _DIM // _SUB
    mesh = plsc.VectorSubcoreMesh(core_axis_name="c", subcore_axis_name="s",
                                  num_cores=_NC, num_subcores=_NS)

    @functools.partial(
        pl.kernel,
        out_type=jax.ShapeDtypeStruct((n_rows, _SUB), jnp.float32),
        mesh=mesh,
        compiler_params=pltpu.CompilerParams(use_tc_tiling_on_sc=False,
                                             needs_layout_passes=False),
        scratch_types=[
            pltpu.VMEM((_NBUF, _CHUNK), jnp.int32),
            pltpu.VMEM((_NBUF, _CHUNK), jnp.int32),
            pltpu.VMEM((_NBUF, _CHUNK, _DIM), jnp.float32),
            pltpu.VMEM((_NBUF, _NBT * _DIM, _PITCH), jnp.float32),
            pltpu.VMEM((_N_PREFIX, _DIM), jnp.float32),
            pltpu.SemaphoreType.DMA,
            pltpu.SemaphoreType.DMA,
            pltpu.SemaphoreType.DMA,
            pltpu.SemaphoreType.DMA,
            pltpu.SemaphoreType.DMA,
            pltpu.SemaphoreType.DMA,
        ],
    )
    def gather_kernel(w_hbm, wn_hbm, idx_hbm, out_hbm,
                      idxo_v, idxc_v, rows_v, stage_v, wn_v,
                      isem0, isem1, gsem0, gsem1, ssem0, ssem1):
        wid = lax.axis_index("s") * _NC + lax.axis_index("c")
        isems = (isem0, isem1)
        gsems = (gsem0, gsem1)
        ssems = (ssem0, ssem1)
        pltpu.sync_copy(wn_hbm, wn_v)
        lanes16 = lax.iota(jnp.int32, _LANES)
        dg_vecs = [dg * _LANES + lanes16 for dg in range(_DIM // _LANES)]
        c_base = wid * n_per_w

        def offsets(c):
            l = c // _BQ
            bq = c % _BQ
            p0 = l * _B + bq * _CHUNK
            r0 = (l * _DT * (_B // _SUB) + bq * _NBT) * 8
            return p0, r0

        def idx_copy(g, b):
            c = c_base + jnp.minimum(g, n_per_w - 1)
            p0, _ = offsets(c)
            return pltpu.make_async_copy(
                idx_hbm.at[pl.ds(p0, _CHUNK)], idxo_v.at[b], isems[b])

        def gather_copies(g, b):
            idxc = idxc_v.at[b]
            rows = rows_v.at[b]
            return [
                pltpu.make_async_copy(
                    w_hbm.at[idxc.at[pl.ds(k * _SUB, _SUB)]],
                    rows.at[pl.ds(k * _SUB, _SUB)],
                    gsems[b],
                )
                for k in range(_NBT)
            ]

        def gather_drain(b):
            idxc = idxc_v.at[b]
            rows = rows_v.at[b]
            for k in range(_NBT):
                pltpu.make_async_copy(
                    w_hbm.at[idxc.at[pl.ds(k * _SUB, _SUB)]],
                    rows.at[pl.ds(k * _SUB, _SUB)],
                    gsems[b]).wait()

        def store_dmas(b, r0):
            stage = stage_v.at[b]
            return [
                pltpu.make_async_copy(
                    stage.at[pl.ds(bt * _DIM + dt * 8, 8), pl.ds(0, _SUB)],
                    out_hbm.at[pl.ds(r0 + dt * (_B // _SUB) * 8 + bt * 8, 8)],
                    ssems[b],
                )
                for dt in range(_DT)
                for bt in range(_NBT)
            ]

        def clamp(b):
            idxo = idxo_v.at[b]
            idxc = idxc_v.at[b]

            @plsc.parallel_loop(0, _CHUNK // _LANES, unroll=4,
                                carry=jnp.bool_(False))
            def has_prefix(j, acc):
                v = idxo[pl.ds(j * _LANES, _LANES)]
                m = v >= _VOCAB
                idxc[pl.ds(j * _LANES, _LANES)] = jnp.where(m, _VOCAB - 1, v)
                return acc | jnp.any(m)

            return has_prefix

        def patch(b, has_prefix):
            idxo = idxo_v.at[b]
            rows = rows_v.at[b]

            @pl.when(has_prefix)
            def _patch():
                def patch_slice(j, acc):
                    v = idxo[pl.ds(j * _LANES, _LANES)]
                    m = v >= _VOCAB

                    @pl.when(jnp.any(m))
                    def _do_patch():
                        e = jnp.where(m, v - _VOCAB, 0)
                        rows16 = j * _LANES + lanes16

                        def col_body(d, cc):
                            colv = jnp.full((_LANES,), d, jnp.int32)
                            vals = plsc.load_gather(wn_v, [e, colv], mask=m)
                            plsc.store_scatter(rows, [rows16, colv], vals,
                                               mask=m)
                            return cc

                        lax.fori_loop(0, _DIM, col_body, jnp.int32(0))

                    return acc

                lax.fori_loop(0, _CHUNK // _LANES, patch_slice, jnp.int32(0))

        def transpose(b):
            rows = rows_v.at[b]
            stage = stage_v.at[b]

            @plsc.parallel_loop(0, _CHUNK, unroll=8)
            def _tr(r):
                bt = lax.shift_right_logical(r, 7)
                bi = lax.bitwise_and(r, _SUB - 1)
                biv = jnp.full((_LANES,), bi, jnp.int32)
                pbase = bt * _DIM
                for dg in range(_DIM // _LANES):
                    pv = pbase + dg_vecs[dg]
                    vals = rows[r, pl.ds(dg * _LANES, _LANES)]
                    plsc.store_scatter(stage, [pv, biv], vals)

        idx_copy(0, 0).start()
        idx_copy(1, 1).start()
        idx_copy(0, 0).wait()
        hp0_init = clamp(0)
        for cp in gather_copies(0, 0):
            cp.start()

        def step_body(s, hp0):
            def half(g, b, hp_b, first):
                gather_drain(b)
                patch(b, hp_b)
                idx_copy(g + 2, b).start()
                idx_copy(g + 1, 1 - b).wait()
                hp_next = clamp(1 - b)
                for cp in gather_copies(g + 1, 1 - b):
                    cp.start()
                _, r0 = offsets(c_base + g)

                @pl.when(jnp.logical_not(first))
                def _drain_prev():
                    for cp in store_dmas(b, r0):
                        cp.wait()

                transpose(b)
                for cp in store_dmas(b, r0):
                    cp.start()
                return hp_next

            g0 = s * _NBUF
            hp1 = half(g0, 0, hp0, s < 1)
            hp0_next = half(g0 + 1, 1, hp1, s < 1)
            return hp0_next

        lax.fori_loop(0, n_per_w // _NBUF, step_body, hp0_init)

        idx_copy(n_per_w + 1, 1).wait()
        gather_drain(0)
        for b in range(_NBUF):
            g = n_per_w - _NBUF + b
            _, r0 = offsets(c_base + g)
            for cp in store_dmas(b, r0):
                cp.wait()

    return gather_kernel


def kernel(input, W, W_new):
    b, l = input.shape
    idx = input.T.reshape(-1).astype(jnp.int32)
    flat = _make_gather(idx.shape[0])(W, W_new, idx)
    out5 = flat.reshape(l, _DIM // 8, b // _SUB, 8, _SUB)
    return out5.transpose(2, 4, 0, 1, 3).reshape(b, l, _DIM)
